# Initial kernel scaffold; baseline (speedup 1.0000x reference)
import jax
import jax.numpy as jnp
from jax.experimental import pallas as pl


def kernel(user_id, item_seq, target_item_pos, target_item_neg, intention_seq, target_intention_pos, target_intention_neg, user_table, item_table, from_intention_table, to_intention_table, timestep_w):
    raise NotImplementedError("write your pallas kernel here")


# speedup vs baseline: 2.2450x; 2.2450x over previous
import functools

import jax
import jax.numpy as jnp
from jax import lax
from jax.experimental import pallas as pl
from jax.experimental.pallas import tpu as pltpu
from jax.experimental.pallas import tpu_sc as plsc

B = 16384
MAXLEN = 200
ATT_LEN = 20
D = 32
N_INT = 1000

NC = 2
NS = 16
L = 16
NW = NC * NS
ROWS_W = B // NW
CH = 256
NCH = ROWS_W // CH
GRP = CH // L


def _body(iseq_hbm, mseq_hbm, uid_hbm, tip_hbm, tin_hbm, itp_hbm, itn_hbm,
          user_hbm, item_hbm, from_hbm, to_hbm, tw_hbm,
          out_hbm,
          from_v, to_v, tw_v, iseq_v, mseq_v, tip_v, tin_v,
          uid_v, itp_v, itn_v, urows, prows, nrows, out_v, sem):
  cid = lax.axis_index("c")
  sid = lax.axis_index("s")
  wid = sid * NC + cid

  pltpu.sync_copy(from_hbm, from_v)
  pltpu.sync_copy(to_hbm, to_v)
  pltpu.sync_copy(tw_hbm, tw_v)

  def do_chunk(h, carry):
    base = wid * ROWS_W + h * CH

    pltpu.sync_copy(iseq_hbm.at[pl.ds(base, CH), :], iseq_v)
    pltpu.sync_copy(mseq_hbm.at[pl.ds(base, CH), :], mseq_v)
    pltpu.sync_copy(tip_hbm.at[pl.ds(base, CH)], tip_v)
    pltpu.sync_copy(tin_hbm.at[pl.ds(base, CH)], tin_v)
    for kk in range(CH // 128):
      pltpu.sync_copy(uid_hbm.at[pl.ds(base + kk * 128, 128)], uid_v.at[kk])
      pltpu.sync_copy(itp_hbm.at[pl.ds(base + kk * 128, 128)], itp_v.at[kk])
      pltpu.sync_copy(itn_hbm.at[pl.ds(base + kk * 128, 128)], itn_v.at[kk])

    copies = []
    for kk in range(CH // 128):
      sl = pl.ds(kk * 128, 128)
      copies.append(pltpu.async_copy(user_hbm.at[uid_v.at[kk]], urows.at[sl, :], sem))
      copies.append(pltpu.async_copy(item_hbm.at[itp_v.at[kk]], prows.at[sl, :], sem))
      copies.append(pltpu.async_copy(item_hbm.at[itn_v.at[kk]], nrows.at[sl, :], sem))
    for cp in copies:
      cp.wait()

    def do_group(g, gcarry):
      rows = lax.iota(jnp.int32, L) + g * L
      acc = [jnp.zeros((L,), jnp.float32) for _ in range(D)]
      for c in range(ATT_LEN):
        col = jnp.full((L,), c, jnp.int32)
        iidx = plsc.load_gather(iseq_v, [rows, col])
        mval = plsc.load_gather(mseq_v, [rows, col])
        wm = jnp.where(mval != 0, tw_v[c, :], jnp.zeros((L,), jnp.float32))
        for d in range(D):
          fv = plsc.load_gather(from_v, [iidx, jnp.full((L,), d, jnp.int32)])
          acc[d] = acc[d] + wm * fv

      tipx = tip_v[pl.ds(g * L, L)]
      tinx = tin_v[pl.ds(g * L, L)]
      sp = jnp.zeros((L,), jnp.float32)
      sn = jnp.zeros((L,), jnp.float32)
      lp = jnp.zeros((L,), jnp.float32)
      ln = jnp.zeros((L,), jnp.float32)
      for d in range(D):
        dcol = jnp.full((L,), d, jnp.int32)
        sp = sp + acc[d] * plsc.load_gather(to_v, [tipx, dcol])
        sn = sn + acc[d] * plsc.load_gather(to_v, [tinx, dcol])
        u = plsc.load_gather(urows, [rows, dcol])
        lp = lp + u * plsc.load_gather(prows, [rows, dcol])
        ln = ln + u * plsc.load_gather(nrows, [rows, dcol])

      o0 = 0.5 * lp + 0.5 * sp
      o1 = 0.5 * ln + 0.5 * sn
      plsc.store_scatter(out_v, [rows, jnp.zeros((L,), jnp.int32)], o0)
      plsc.store_scatter(out_v, [rows, jnp.ones((L,), jnp.int32)], o1)
      return gcarry

    lax.fori_loop(0, GRP, do_group, 0)
    pltpu.sync_copy(out_v, out_hbm.at[pl.ds(base, CH), :])
    return carry

  lax.fori_loop(0, NCH, do_chunk, 0)


@jax.jit
def _run(iseq_s, mseq_s, uid, tip, tin, itp, itn,
         user_table, item_table, from_t, to_t, tw_prep):
  mesh = plsc.VectorSubcoreMesh(core_axis_name="c", subcore_axis_name="s")
  f = pl.kernel(
      _body,
      out_type=jax.ShapeDtypeStruct((B, 2), jnp.float32),
      mesh=mesh,
      scratch_types=[
          pltpu.VMEM((N_INT, D), jnp.float32),
          pltpu.VMEM((N_INT, D), jnp.float32),
          pltpu.VMEM((ATT_LEN, L), jnp.float32),
          pltpu.VMEM((CH, ATT_LEN), jnp.int32),
          pltpu.VMEM((CH, ATT_LEN), jnp.int32),
          pltpu.VMEM((CH,), jnp.int32),
          pltpu.VMEM((CH,), jnp.int32),
          pltpu.VMEM((CH // 128, 128), jnp.int32),
          pltpu.VMEM((CH // 128, 128), jnp.int32),
          pltpu.VMEM((CH // 128, 128), jnp.int32),
          pltpu.VMEM((CH, D), jnp.float32),
          pltpu.VMEM((CH, D), jnp.float32),
          pltpu.VMEM((CH, D), jnp.float32),
          pltpu.VMEM((CH, 2), jnp.float32),
          pltpu.SemaphoreType.DMA,
      ],
      compiler_params=pltpu.CompilerParams(
          needs_layout_passes=False, use_tc_tiling_on_sc=False),
  )
  return f(iseq_s, mseq_s, uid, tip, tin, itp, itn,
           user_table, item_table, from_t, to_t, tw_prep)


def kernel(user_id, item_seq, target_item_pos, target_item_neg,
           intention_seq, target_intention_pos, target_intention_neg,
           user_table, item_table, from_intention_table,
           to_intention_table, timestep_w):
  iseq_s = intention_seq[:, MAXLEN - ATT_LEN:]
  mseq_s = item_seq[:, MAXLEN - ATT_LEN:]
  uid = user_id[:, 0]
  tip = target_intention_pos[:, 0]
  tin = target_intention_neg[:, 0]
  itp = target_item_pos[:, 0]
  itn = target_item_neg[:, 0]
  tw_prep = jnp.broadcast_to(timestep_w[::-1][:, None], (ATT_LEN, L))
  return _run(iseq_s, mseq_s, uid, tip, tin, itp, itn,
              user_table, item_table, from_intention_table,
              to_intention_table, tw_prep)


# speedup vs baseline: 2.9833x; 1.3289x over previous
import jax
import jax.numpy as jnp
from jax import lax
from jax.experimental import pallas as pl
from jax.experimental.pallas import tpu as pltpu
from jax.experimental.pallas import tpu_sc as plsc

B = 16384
MAXLEN = 200
ATT_LEN = 20
D = 32
N_INT = 1000

NC = 2
NS = 16
L = 16
NW = NC * NS
ROWS_W = B // NW
SUB = 64
NSUB = ROWS_W // SUB
GR = 8
SEQP = SUB + L - GR


def _body(iseq_hbm, mseq_hbm, uid_hbm, tip_hbm, tin_hbm, itp_hbm, itn_hbm,
          user_hbm, item_hbm, from_hbm, to_hbm, tw_hbm,
          out_hbm,
          from_v, to_v, tw_v, iseq_v, mseq_v, tip_v, tin_v,
          uid_v, itp_v, itn_v, idxu_v, idxp_v, idxn_v,
          ustage, pstage, nstage, pooled_v, out_v, sem):
  cid = lax.axis_index("c")
  sid = lax.axis_index("s")
  wid = sid * NC + cid

  pltpu.sync_copy(from_hbm, from_v)
  pltpu.sync_copy(to_hbm, to_v)
  pltpu.sync_copy(tw_hbm, tw_v)

  viota = lax.iota(jnp.int32, L)
  zf = jnp.zeros((L,), jnp.float32)

  def do_sub(h, carry):
    base = wid * ROWS_W + h * SUB

    pltpu.sync_copy(iseq_hbm.at[:, pl.ds(base, SUB)], iseq_v.at[:, 0:SUB])
    pltpu.sync_copy(mseq_hbm.at[:, pl.ds(base, SUB)], mseq_v.at[:, 0:SUB])
    pltpu.sync_copy(tip_hbm.at[pl.ds(base, SUB)], tip_v)
    pltpu.sync_copy(tin_hbm.at[pl.ds(base, SUB)], tin_v)
    pltpu.sync_copy(uid_hbm.at[pl.ds(base, SUB)], uid_v)
    pltpu.sync_copy(itp_hbm.at[pl.ds(base, SUB)], itp_v)
    pltpu.sync_copy(itn_hbm.at[pl.ds(base, SUB)], itn_v)

    for j in range(SUB // L):
      sl = pl.ds(j * L, L)
      idxu_v[sl] = lax.shift_right_logical(uid_v[sl], 2)
      idxp_v[sl] = lax.shift_right_logical(itp_v[sl], 2)
      idxn_v[sl] = lax.shift_right_logical(itn_v[sl], 2)

    cps = [
        pltpu.async_copy(user_hbm.at[idxu_v], ustage, sem),
        pltpu.async_copy(item_hbm.at[idxp_v], pstage, sem),
        pltpu.async_copy(item_hbm.at[idxn_v], nstage, sem),
    ]

    def pool_group(g, pcarry):
      accs = [zf] * (2 * GR)
      for c in range(ATT_LEN):
        iidxv = iseq_v[c, pl.ds(g * GR, L)]
        mvalv = mseq_v[c, pl.ds(g * GR, L)]
        wmv = jnp.where(mvalv != 0, tw_v[c, :], zf)
        for l in range(GR):
          s = iidxv[l]
          f0 = from_v[s, pl.ds(0, L)]
          f1 = from_v[s, pl.ds(L, L)]
          wmb = jnp.broadcast_to(wmv[l], (L,))
          accs[2 * l] = accs[2 * l] + wmb * f0
          accs[2 * l + 1] = accs[2 * l + 1] + wmb * f1
      for l in range(GR):
        r = g * GR + l
        pooled_v[r, pl.ds(0, L)] = accs[2 * l]
        pooled_v[r, pl.ds(L, L)] = accs[2 * l + 1]
      return pcarry

    lax.fori_loop(0, SUB // GR, pool_group, 0)

    for cp in cps:
      cp.wait()

    def dot_group(g, dcarry):
      rows = viota + g * L
      sl = pl.ds(g * L, L)
      tipx = tip_v[sl]
      tinx = tin_v[sl]
      o32u = lax.shift_left(jnp.bitwise_and(uid_v[sl], 3), 5)
      o32p = lax.shift_left(jnp.bitwise_and(itp_v[sl], 3), 5)
      o32n = lax.shift_left(jnp.bitwise_and(itn_v[sl], 3), 5)
      sp = zf
      sn = zf
      lp = zf
      ln = zf
      for k in range(D):
        dvec = jnp.bitwise_and(viota + k, D - 1)
        pv = plsc.load_gather(pooled_v, [rows, dvec])
        sp = sp + pv * plsc.load_gather(to_v, [tipx, dvec])
        sn = sn + pv * plsc.load_gather(to_v, [tinx, dvec])
        u = plsc.load_gather(ustage, [rows, o32u + dvec])
        lp = lp + u * plsc.load_gather(pstage, [rows, o32p + dvec])
        ln = ln + u * plsc.load_gather(nstage, [rows, o32n + dvec])
      o0 = 0.5 * lp + 0.5 * sp
      o1 = 0.5 * ln + 0.5 * sn
      plsc.store_scatter(out_v, [rows, jnp.zeros((L,), jnp.int32)], o0)
      plsc.store_scatter(out_v, [rows, jnp.ones((L,), jnp.int32)], o1)
      return dcarry

    lax.fori_loop(0, SUB // L, dot_group, 0)

    pltpu.sync_copy(out_v, out_hbm.at[pl.ds(base, SUB), :])
    return carry

  lax.fori_loop(0, NSUB, do_sub, 0)


@jax.jit
def _run(iseq_t, mseq_t, uid, tip, tin, itp, itn,
         user128, item128, from_t, to_t, tw_prep):
  mesh = plsc.VectorSubcoreMesh(core_axis_name="c", subcore_axis_name="s")
  f = pl.kernel(
      _body,
      out_type=jax.ShapeDtypeStruct((B, 2), jnp.float32),
      mesh=mesh,
      scratch_types=[
          pltpu.VMEM((N_INT, D), jnp.float32),
          pltpu.VMEM((N_INT, D), jnp.float32),
          pltpu.VMEM((ATT_LEN, L), jnp.float32),
          pltpu.VMEM((ATT_LEN, SEQP), jnp.int32),
          pltpu.VMEM((ATT_LEN, SEQP), jnp.int32),
          pltpu.VMEM((SUB,), jnp.int32),
          pltpu.VMEM((SUB,), jnp.int32),
          pltpu.VMEM((SUB,), jnp.int32),
          pltpu.VMEM((SUB,), jnp.int32),
          pltpu.VMEM((SUB,), jnp.int32),
          pltpu.VMEM((SUB,), jnp.int32),
          pltpu.VMEM((SUB,), jnp.int32),
          pltpu.VMEM((SUB,), jnp.int32),
          pltpu.VMEM((SUB, 128), jnp.float32),
          pltpu.VMEM((SUB, 128), jnp.float32),
          pltpu.VMEM((SUB, 128), jnp.float32),
          pltpu.VMEM((SUB, D), jnp.float32),
          pltpu.VMEM((SUB, 2), jnp.float32),
          pltpu.SemaphoreType.DMA,
      ],
      compiler_params=pltpu.CompilerParams(
          needs_layout_passes=False, use_tc_tiling_on_sc=False),
  )
  return f(iseq_t, mseq_t, uid, tip, tin, itp, itn,
           user128, item128, from_t, to_t, tw_prep)


def kernel(user_id, item_seq, target_item_pos, target_item_neg,
           intention_seq, target_intention_pos, target_intention_neg,
           user_table, item_table, from_intention_table,
           to_intention_table, timestep_w):
  iseq_t = intention_seq[:, MAXLEN - ATT_LEN:].T
  mseq_t = item_seq[:, MAXLEN - ATT_LEN:].T
  uid = user_id[:, 0]
  tip = target_intention_pos[:, 0]
  tin = target_intention_neg[:, 0]
  itp = target_item_pos[:, 0]
  itn = target_item_neg[:, 0]
  user128 = user_table.reshape(user_table.shape[0] // 4, 128)
  item128 = item_table.reshape(item_table.shape[0] // 4, 128)
  tw_prep = jnp.broadcast_to(timestep_w[::-1][:, None], (ATT_LEN, L))
  return _run(iseq_t, mseq_t, uid, tip, tin, itp, itn,
              user128, item128, from_intention_table,
              to_intention_table, tw_prep)
